# baseline (speedup 1.0000x reference)
import functools

import jax
import jax.numpy as jnp
from jax import lax
from jax.experimental import pallas as pl
from jax.experimental.pallas import tpu as pltpu
from jax.experimental.pallas import tpu_sc as plsc

NC = 2
NS = 16
CH = 128


def _tc_table_kernel(x_ref, w_ref, src_ref, et_ref, tab_ref, gidx_ref, *,
                     plane_rows):
    tab_ref[0] = jnp.dot(x_ref[...], w_ref[...],
                         preferred_element_type=jnp.float32)

    @pl.when(pl.program_id(0) == 0)
    def _():
        et = et_ref[...]
        gidx_ref[...] = ((et >> 3) * plane_rows + (src_ref[...] << 3)
                         + (et & 7))


def _tc_combine_kernel(p_ref, b_ref, o_ref):
    o_ref[...] = p_ref[0] + p_ref[1] + b_ref[...]


def _sc_scatter_body(table_hbm, gidx_hbm, dst_hbm, zrows_hbm, out_hbm,
                     gidx_v, dst_v, bufs, vout, acc_sh,
                     gsems, ssems, *, nchunk, rows_per_tile, nbuf):
    c = lax.axis_index("c")
    s = lax.axis_index("s")
    wid = s * NC + c

    pltpu.sync_copy(gidx_hbm.at[wid], gidx_v)
    pltpu.sync_copy(dst_hbm.at[wid], dst_v)

    pltpu.sync_copy(zrows_hbm, vout)
    pltpu.sync_copy(vout, acc_sh.at[pl.ds(s * rows_per_tile, rows_per_tile)])
    plsc.subcore_barrier()

    def start_gather(j, b):
        pltpu.async_copy(table_hbm.at[gidx_v.at[j]], bufs.at[b], gsems.at[b])

    def wait_gather(b):
        pltpu.make_async_copy(table_hbm.at[gidx_v.at[0]], bufs.at[b],
                              gsems.at[b]).wait()

    def start_scatter(j, b):
        pltpu.async_copy(bufs.at[b], acc_sh.at[dst_v.at[j]], ssems.at[b],
                         add=True)

    def wait_scatter(b):
        pltpu.make_async_copy(bufs.at[b], acc_sh.at[dst_v.at[0]],
                              ssems.at[b]).wait()

    for b in range(nbuf):
        start_gather(b, b)

    def body(i, carry):
        j0 = i * nbuf
        for b in range(nbuf):
            wait_gather(b)
            start_scatter(j0 + b, b)
        for b in range(nbuf):
            @pl.when(j0 + nbuf + b < nchunk)
            def _(b=b):
                wait_scatter(b)
                start_gather(j0 + nbuf + b, b)
        return carry

    lax.fori_loop(0, nchunk // nbuf, body, 0)
    for b in range(nbuf):
        wait_scatter(b)
    plsc.subcore_barrier()

    pltpu.sync_copy(acc_sh.at[pl.ds(s * rows_per_tile, rows_per_tile)], vout)
    pltpu.sync_copy(vout, out_hbm.at[c, pl.ds(s * rows_per_tile, rows_per_tile)])


def kernel(x, edge_index, edge_type, W, loop_weight, bias):
    n, d_in = x.shape
    e = edge_type.shape[0]
    r = W.shape[0]
    d_out = W.shape[2]
    f32 = jnp.float32

    slot = 16
    r_pad = ((r + 1 + 7) // 8) * 8
    n_acc = ((n + 1 + 8 * NS - 1) // (8 * NS)) * (8 * NS)
    rows_per_tile = n_acc // NS
    dummy = n
    nw = NC * NS
    e_full = e + n
    nbuf = 4
    nchunk = -(-e_full // (nw * CH))
    nchunk = ((nchunk + nbuf - 1) // nbuf) * nbuf
    e_pad = nw * nchunk * CH
    ep_rows = e_pad // 128

    w_full = jnp.concatenate([W, loop_weight[None]], axis=0)
    w_pad = jnp.zeros((r_pad, d_in, slot), f32).at[:r + 1, :, :d_out].set(w_full)
    w_cat = w_pad.transpose(1, 0, 2).reshape(d_in, r_pad * slot)

    ar = jnp.arange(n, dtype=jnp.int32)
    src_f = jnp.concatenate([edge_index[0], ar])
    et_f = jnp.concatenate([edge_type, jnp.full((n,), r, jnp.int32)])
    dst_f = jnp.concatenate([edge_index[1], ar])
    pad = e_pad - e_full
    src_r = jnp.pad(src_f, (0, pad)).reshape(ep_rows, 128)
    et_r = jnp.pad(et_f, (0, pad)).reshape(ep_rows, 128)
    dst_r = jnp.pad(dst_f, (0, pad), constant_values=dummy).reshape(
        nw, nchunk, CH)

    zrows = jnp.zeros((rows_per_tile, slot), f32)
    brow = jnp.concatenate([bias.astype(f32), jnp.zeros((slot - d_out,), f32)])
    bias_flat = jnp.broadcast_to(brow, (n_acc, slot)).reshape(
        n_acc * slot // 128, 128)

    planes = r_pad * slot // 128
    plane_rows = n * 128 // slot
    table, gidx = pl.pallas_call(
        functools.partial(_tc_table_kernel, plane_rows=plane_rows),
        grid=(planes,),
        in_specs=[
            pl.BlockSpec((n, d_in), lambda g: (0, 0)),
            pl.BlockSpec((d_in, 128), lambda g: (0, g)),
            pl.BlockSpec((ep_rows, 128), lambda g: (0, 0)),
            pl.BlockSpec((ep_rows, 128), lambda g: (0, 0)),
        ],
        out_specs=[
            pl.BlockSpec((1, n, 128), lambda g: (g, 0, 0)),
            pl.BlockSpec((ep_rows, 128), lambda g: (0, 0)),
        ],
        out_shape=[
            jax.ShapeDtypeStruct((planes, n, 128), f32),
            jax.ShapeDtypeStruct((ep_rows, 128), jnp.int32),
        ],
    )(x, w_cat, src_r, et_r)

    mesh = plsc.VectorSubcoreMesh(core_axis_name="c", subcore_axis_name="s",
                                  num_cores=NC, num_subcores=NS)
    sc = pl.kernel(
        functools.partial(_sc_scatter_body, nchunk=nchunk,
                          rows_per_tile=rows_per_tile, nbuf=nbuf),
        out_type=jax.ShapeDtypeStruct((NC, n_acc, slot), f32),
        mesh=mesh,
        compiler_params=pltpu.CompilerParams(use_tc_tiling_on_sc=False),
        scratch_types=[
            pltpu.VMEM((nchunk, CH), jnp.int32),
            pltpu.VMEM((nchunk, CH), jnp.int32),
            pltpu.VMEM((nbuf, CH, slot), f32),
            pltpu.VMEM((rows_per_tile, slot), f32),
            pltpu.VMEM_SHARED((n_acc, slot), f32),
            pltpu.SemaphoreType.DMA((nbuf,)),
            pltpu.SemaphoreType.DMA((nbuf,)),
        ],
    )
    partials = sc(table.reshape(n * r_pad, slot),
                  gidx.reshape(nw, nchunk, CH), dst_r, zrows)

    flat_rows = n_acc * slot // 128
    out_flat = pl.pallas_call(
        _tc_combine_kernel,
        grid=(1,),
        in_specs=[
            pl.BlockSpec((NC, flat_rows, 128), lambda g: (0, 0, 0)),
            pl.BlockSpec((flat_rows, 128), lambda g: (0, 0)),
        ],
        out_specs=pl.BlockSpec((flat_rows, 128), lambda g: (0, 0)),
        out_shape=jax.ShapeDtypeStruct((flat_rows, 128), f32),
    )(partials.reshape(NC, flat_rows, 128), bias_flat)

    return out_flat.reshape(n_acc, slot)[:n, :d_out]


# speedup vs baseline: 1.4056x; 1.4056x over previous
import functools

import jax
import jax.numpy as jnp
from jax import lax
from jax.experimental import pallas as pl
from jax.experimental.pallas import tpu as pltpu
from jax.experimental.pallas import tpu_sc as plsc

NC = 2
NS = 16
CH = 128


def _tc_table_kernel(x_ref, w_ref, src_ref, et_ref, tab_ref, gidx_ref, *,
                     plane_rows):
    tab_ref[0] = jnp.dot(x_ref[...], w_ref[...],
                         preferred_element_type=jnp.float32)

    @pl.when(pl.program_id(0) == 0)
    def _():
        et = et_ref[...]
        gidx_ref[...] = ((et >> 3) * plane_rows + (src_ref[...] << 3)
                         + (et & 7))


def _tc_combine_kernel(p_ref, b_ref, o_ref):
    o_ref[...] = p_ref[0] + p_ref[1] + b_ref[...]


def _sc_scatter_body(table_hbm, gidx_hbm, dst_hbm, zrows_hbm, out_hbm,
                     gidx_v, dst_v, bufs, vout, acc_sh,
                     gsems, ssems, *, nchunk, rows_per_tile, nbuf):
    c = lax.axis_index("c")
    s = lax.axis_index("s")
    wid = s * NC + c

    pltpu.sync_copy(gidx_hbm.at[wid], gidx_v)
    pltpu.sync_copy(dst_hbm.at[wid], dst_v)

    pltpu.sync_copy(zrows_hbm, vout)
    pltpu.sync_copy(vout, acc_sh.at[pl.ds(s * rows_per_tile, rows_per_tile)])
    plsc.subcore_barrier()

    def start_gather(j, b):
        pltpu.async_copy(table_hbm.at[gidx_v.at[j]], bufs.at[b], gsems.at[b])

    def wait_gather(b):
        pltpu.make_async_copy(table_hbm.at[gidx_v.at[0]], bufs.at[b],
                              gsems.at[b]).wait()

    def start_scatter(j, b):
        pltpu.async_copy(bufs.at[b], acc_sh.at[dst_v.at[j]], ssems.at[b],
                         add=True)

    def wait_scatter(b):
        pltpu.make_async_copy(bufs.at[b], acc_sh.at[dst_v.at[0]],
                              ssems.at[b]).wait()

    start_gather(0, 0)
    start_gather(1, 1)
    wait_gather(0)
    start_scatter(0, 0)
    start_gather(2, 2)

    def body(i, carry):
        for t in range(3):
            j = 3 * i + 1 + t
            b = (1 + t) % 3
            wait_gather(b)
            wait_scatter((b + 2) % 3)
            start_scatter(j, b)

            @pl.when(j + 2 < nchunk)
            def _(j=j, b=b):
                start_gather(j + 2, (b + 2) % 3)
        return carry

    lax.fori_loop(0, (nchunk - 1) // 3, body, 0)
    wait_scatter((nchunk - 1) % 3)
    plsc.subcore_barrier()

    pltpu.sync_copy(acc_sh.at[pl.ds(s * rows_per_tile, rows_per_tile)], vout)
    pltpu.sync_copy(vout, out_hbm.at[c, pl.ds(s * rows_per_tile, rows_per_tile)])


def kernel(x, edge_index, edge_type, W, loop_weight, bias):
    n, d_in = x.shape
    e = edge_type.shape[0]
    r = W.shape[0]
    d_out = W.shape[2]
    f32 = jnp.float32

    slot = 16
    r_pad = ((r + 1 + 7) // 8) * 8
    n_acc = ((n + 1 + 8 * NS - 1) // (8 * NS)) * (8 * NS)
    rows_per_tile = n_acc // NS
    dummy = n
    nw = NC * NS
    e_full = e + n
    nbuf = 3
    nchunk = -(-e_full // (nw * CH))
    while (nchunk - 1) % 3:
        nchunk += 1
    e_pad = nw * nchunk * CH
    ep_rows = e_pad // 128

    w_full = jnp.concatenate([W, loop_weight[None]], axis=0)
    w_pad = jnp.zeros((r_pad, d_in, slot), f32).at[:r + 1, :, :d_out].set(w_full)
    w_cat = w_pad.transpose(1, 0, 2).reshape(d_in, r_pad * slot)

    ar = jnp.arange(n, dtype=jnp.int32)
    src_f = jnp.concatenate([edge_index[0], ar])
    et_f = jnp.concatenate([edge_type, jnp.full((n,), r, jnp.int32)])
    dst_f = jnp.concatenate([edge_index[1], ar])
    pad = e_pad - e_full
    src_r = jnp.pad(src_f, (0, pad)).reshape(ep_rows, 128)
    et_r = jnp.pad(et_f, (0, pad)).reshape(ep_rows, 128)
    dst_r = jnp.pad(dst_f, (0, pad), constant_values=dummy).reshape(
        nw, nchunk, CH)

    zrows = jnp.zeros((rows_per_tile, slot), f32)
    brow = jnp.concatenate([bias.astype(f32), jnp.zeros((slot - d_out,), f32)])
    bias_flat = jnp.broadcast_to(brow, (n_acc, slot)).reshape(
        n_acc * slot // 128, 128)

    planes = r_pad * slot // 128
    plane_rows = n * 128 // slot
    table, gidx = pl.pallas_call(
        functools.partial(_tc_table_kernel, plane_rows=plane_rows),
        grid=(planes,),
        in_specs=[
            pl.BlockSpec((n, d_in), lambda g: (0, 0)),
            pl.BlockSpec((d_in, 128), lambda g: (0, g)),
            pl.BlockSpec((ep_rows, 128), lambda g: (0, 0)),
            pl.BlockSpec((ep_rows, 128), lambda g: (0, 0)),
        ],
        out_specs=[
            pl.BlockSpec((1, n, 128), lambda g: (g, 0, 0)),
            pl.BlockSpec((ep_rows, 128), lambda g: (0, 0)),
        ],
        out_shape=[
            jax.ShapeDtypeStruct((planes, n, 128), f32),
            jax.ShapeDtypeStruct((ep_rows, 128), jnp.int32),
        ],
    )(x, w_cat, src_r, et_r)

    mesh = plsc.VectorSubcoreMesh(core_axis_name="c", subcore_axis_name="s",
                                  num_cores=NC, num_subcores=NS)
    sc = pl.kernel(
        functools.partial(_sc_scatter_body, nchunk=nchunk,
                          rows_per_tile=rows_per_tile, nbuf=nbuf),
        out_type=jax.ShapeDtypeStruct((NC, n_acc, slot), f32),
        mesh=mesh,
        compiler_params=pltpu.CompilerParams(use_tc_tiling_on_sc=False),
        scratch_types=[
            pltpu.VMEM((nchunk, CH), jnp.int32),
            pltpu.VMEM((nchunk, CH), jnp.int32),
            pltpu.VMEM((nbuf, CH, slot), f32),
            pltpu.VMEM((rows_per_tile, slot), f32),
            pltpu.VMEM_SHARED((n_acc, slot), f32),
            pltpu.SemaphoreType.DMA((nbuf,)),
            pltpu.SemaphoreType.DMA((nbuf,)),
        ],
    )
    partials = sc(table.reshape(n * r_pad, slot),
                  gidx.reshape(nw, nchunk, CH), dst_r, zrows)

    flat_rows = n_acc * slot // 128
    out_flat = pl.pallas_call(
        _tc_combine_kernel,
        grid=(1,),
        in_specs=[
            pl.BlockSpec((NC, flat_rows, 128), lambda g: (0, 0, 0)),
            pl.BlockSpec((flat_rows, 128), lambda g: (0, 0)),
        ],
        out_specs=pl.BlockSpec((flat_rows, 128), lambda g: (0, 0)),
        out_shape=jax.ShapeDtypeStruct((flat_rows, 128), f32),
    )(partials.reshape(NC, flat_rows, 128), bias_flat)

    return out_flat.reshape(n_acc, slot)[:n, :d_out]


# speedup vs baseline: 1.5557x; 1.1068x over previous
import functools

import jax
import jax.numpy as jnp
from jax import lax
from jax.experimental import pallas as pl
from jax.experimental.pallas import tpu as pltpu
from jax.experimental.pallas import tpu_sc as plsc

NC = 2
NS = 16
CH = 128


def _tc_table_kernel(x_ref, w_ref, src_ref, et_ref, tab_ref, gidx_ref, *,
                     plane_rows, spp):
    tab_ref[0] = jnp.dot(x_ref[...], w_ref[...],
                         preferred_element_type=jnp.float32)

    @pl.when(pl.program_id(0) == 0)
    def _():
        et = et_ref[...]
        gidx_ref[...] = ((et // spp) * plane_rows + src_ref[...] * spp
                         + (et % spp))


def _tc_combine_kernel(p_ref, b_ref, o_ref):
    o_ref[...] = p_ref[0] + p_ref[1] + b_ref[...]


def _sc_scatter_body(table_hbm, gidx_hbm, dst_hbm, zrows_hbm, out_hbm,
                     gidx_v, dst_v, bufs, vout, acc_sh,
                     gsems, ssems, *, nchunk, rows_per_tile, nbuf):
    c = lax.axis_index("c")
    s = lax.axis_index("s")
    wid = s * NC + c

    pltpu.sync_copy(gidx_hbm.at[wid], gidx_v)
    pltpu.sync_copy(dst_hbm.at[wid], dst_v)

    pltpu.sync_copy(zrows_hbm, vout)
    pltpu.sync_copy(vout, acc_sh.at[pl.ds(s * rows_per_tile, rows_per_tile)])
    plsc.subcore_barrier()

    def start_gather(j, b):
        pltpu.async_copy(table_hbm.at[gidx_v.at[j]], bufs.at[b], gsems.at[b])

    def wait_gather(b):
        pltpu.make_async_copy(table_hbm.at[gidx_v.at[0]], bufs.at[b],
                              gsems.at[b]).wait()

    def start_scatter(j, b):
        pltpu.async_copy(bufs.at[b], acc_sh.at[dst_v.at[j]], ssems.at[b],
                         add=True)

    def wait_scatter(b):
        pltpu.make_async_copy(bufs.at[b], acc_sh.at[dst_v.at[0]],
                              ssems.at[b]).wait()

    start_gather(0, 0)
    start_gather(1, 1)
    wait_gather(0)
    start_scatter(0, 0)
    start_gather(2, 2)

    def body(i, carry):
        for t in range(3):
            j = 3 * i + 1 + t
            b = (1 + t) % 3
            wait_gather(b)
            wait_scatter((b + 2) % 3)
            start_scatter(j, b)

            @pl.when(j + 2 < nchunk)
            def _(j=j, b=b):
                start_gather(j + 2, (b + 2) % 3)
        return carry

    lax.fori_loop(0, (nchunk - 1) // 3, body, 0)
    wait_scatter((nchunk - 1) % 3)
    plsc.subcore_barrier()

    pltpu.sync_copy(acc_sh.at[pl.ds(s * rows_per_tile, rows_per_tile)], vout)
    pltpu.sync_copy(vout, out_hbm.at[c, pl.ds(s * rows_per_tile, rows_per_tile)])


def kernel(x, edge_index, edge_type, W, loop_weight, bias):
    n, d_in = x.shape
    e = edge_type.shape[0]
    r = W.shape[0]
    d_out = W.shape[2]
    f32 = jnp.float32

    slot = 8
    spp = 128 // slot
    r_pad = ((r + 1 + spp - 1) // spp) * spp
    n_acc = ((n + 1 + 8 * NS - 1) // (8 * NS)) * (8 * NS)
    rows_per_tile = n_acc // NS
    dummy = n
    nw = NC * NS
    e_full = e + n
    nbuf = 3
    nchunk = -(-e_full // (nw * CH))
    while (nchunk - 1) % 3:
        nchunk += 1
    e_pad = nw * nchunk * CH
    ep_rows = e_pad // 128

    w_full = jnp.concatenate([W, loop_weight[None]], axis=0)
    w_pad = jnp.zeros((r_pad, d_in, slot), f32).at[:r + 1, :, :d_out].set(w_full)
    w_cat = w_pad.transpose(1, 0, 2).reshape(d_in, r_pad * slot)

    ar = jnp.arange(n, dtype=jnp.int32)
    src_f = jnp.concatenate([edge_index[0], ar])
    et_f = jnp.concatenate([edge_type, jnp.full((n,), r, jnp.int32)])
    dst_f = jnp.concatenate([edge_index[1], ar])
    pad = e_pad - e_full
    src_r = jnp.pad(src_f, (0, pad)).reshape(ep_rows, 128)
    et_r = jnp.pad(et_f, (0, pad)).reshape(ep_rows, 128)
    dst_r = jnp.pad(dst_f, (0, pad), constant_values=dummy).reshape(
        nw, nchunk, CH)

    zrows = jnp.zeros((rows_per_tile, slot), f32)
    brow = jnp.concatenate([bias.astype(f32), jnp.zeros((slot - d_out,), f32)])
    bias_flat = jnp.broadcast_to(brow, (n_acc, slot)).reshape(
        n_acc * slot // 128, 128)

    planes = r_pad * slot // 128
    plane_rows = n * 128 // slot
    table, gidx = pl.pallas_call(
        functools.partial(_tc_table_kernel, plane_rows=plane_rows, spp=spp),
        grid=(planes,),
        in_specs=[
            pl.BlockSpec((n, d_in), lambda g: (0, 0)),
            pl.BlockSpec((d_in, 128), lambda g: (0, g)),
            pl.BlockSpec((ep_rows, 128), lambda g: (0, 0)),
            pl.BlockSpec((ep_rows, 128), lambda g: (0, 0)),
        ],
        out_specs=[
            pl.BlockSpec((1, n, 128), lambda g: (g, 0, 0)),
            pl.BlockSpec((ep_rows, 128), lambda g: (0, 0)),
        ],
        out_shape=[
            jax.ShapeDtypeStruct((planes, n, 128), f32),
            jax.ShapeDtypeStruct((ep_rows, 128), jnp.int32),
        ],
    )(x, w_cat, src_r, et_r)

    mesh = plsc.VectorSubcoreMesh(core_axis_name="c", subcore_axis_name="s",
                                  num_cores=NC, num_subcores=NS)
    sc = pl.kernel(
        functools.partial(_sc_scatter_body, nchunk=nchunk,
                          rows_per_tile=rows_per_tile, nbuf=nbuf),
        out_type=jax.ShapeDtypeStruct((NC, n_acc, slot), f32),
        mesh=mesh,
        compiler_params=pltpu.CompilerParams(use_tc_tiling_on_sc=False),
        scratch_types=[
            pltpu.VMEM((nchunk, CH), jnp.int32),
            pltpu.VMEM((nchunk, CH), jnp.int32),
            pltpu.VMEM((nbuf, CH, slot), f32),
            pltpu.VMEM((rows_per_tile, slot), f32),
            pltpu.VMEM_SHARED((n_acc, slot), f32),
            pltpu.SemaphoreType.DMA((nbuf,)),
            pltpu.SemaphoreType.DMA((nbuf,)),
        ],
    )
    partials = sc(table.reshape(n * r_pad, slot),
                  gidx.reshape(nw, nchunk, CH), dst_r, zrows)

    flat_rows = n_acc * slot // 128
    out_flat = pl.pallas_call(
        _tc_combine_kernel,
        grid=(1,),
        in_specs=[
            pl.BlockSpec((NC, flat_rows, 128), lambda g: (0, 0, 0)),
            pl.BlockSpec((flat_rows, 128), lambda g: (0, 0)),
        ],
        out_specs=pl.BlockSpec((flat_rows, 128), lambda g: (0, 0)),
        out_shape=jax.ShapeDtypeStruct((flat_rows, 128), f32),
    )(partials.reshape(NC, flat_rows, 128), bias_flat)

    return out_flat.reshape(n_acc, slot)[:n, :d_out]
